# baseline (speedup 1.0000x reference)
import functools

import jax
import jax.numpy as jnp
from jax import lax
from jax.experimental import pallas as pl
from jax.experimental.pallas import tpu as pltpu
from jax.experimental.pallas import tpu_sc as plsc

B, Q, C = 32, 5000, 91
N = Q * C
TOPK = 100
KPAD = 112
SCORE_THRESHOLD = 0.05

NVEC = Q // 16
NBLK = NVEC // 8
TAILQ = Q - 16

NEG = float("-inf")
BIGI = 2**31 - 1


def _ffs(mask, iota):
    del iota
    return plsc.all_reduce_ffs(mask)[0]


def _any(mask):
    return plsc.all_reduce_population_count(mask)[0] > 0


def _sc_call(logits_flat, boxes_flat, scale16):
    mesh = plsc.VectorSubcoreMesh(core_axis_name="c", subcore_axis_name="s")

    @functools.partial(
        pl.kernel,
        mesh=mesh,
        compiler_params=pltpu.CompilerParams(needs_layout_passes=False),
        out_type=[
            jax.ShapeDtypeStruct((B, KPAD), jnp.float32),
            jax.ShapeDtypeStruct((B, KPAD), jnp.int32),
            jax.ShapeDtypeStruct((B, 4 * KPAD), jnp.float32),
        ],
        scratch_types=[
            pltpu.VMEM((Q,), jnp.float32),
            pltpu.VMEM((Q,), jnp.float32),
            pltpu.VMEM((KPAD,), jnp.float32),
            pltpu.VMEM((KPAD,), jnp.int32),
            pltpu.VMEM((KPAD,), jnp.float32),
            pltpu.VMEM((KPAD,), jnp.int32),
            pltpu.VMEM((KPAD,), jnp.int32),
            pltpu.VMEM((4, Q), jnp.float32),
            pltpu.VMEM((KPAD,), jnp.float32),
            pltpu.VMEM((KPAD,), jnp.int32),
            pltpu.VMEM((4 * KPAD,), jnp.float32),
            pltpu.VMEM((16,), jnp.float32),
            pltpu.SMEM((1,), jnp.float32),
            pltpu.SMEM((1,), jnp.int32),
            pltpu.SMEM((8,), jnp.float32),
            pltpu.SemaphoreType.DMA,
            pltpu.SemaphoreType.DMA,
        ],
    )
    def body(logits_hbm, boxes_hbm, scale_hbm,
             out_s, out_l, out_x,
             bufa, bufb, topv, topi, srtv, srti, qidx, boxtab, scv, lbv,
             xyv, s16, smin, spos, svm, sema, semb):
        b = lax.axis_index("s") * 2 + lax.axis_index("c")
        iota = lax.iota(jnp.int32, 16)

        PIN = jnp.float32(3e38)
        for t in range(7):
            init = jnp.full((16,), NEG, jnp.float32)
            if t == 6:
                init = jnp.where(iota < 4, jnp.float32(NEG), PIN)
            topv[pl.ds(16 * t, 16)] = init
            topi[pl.ds(16 * t, 16)] = jnp.zeros((16,), jnp.int32)
            srtv[pl.ds(16 * t, 16)] = jnp.full((16,), NEG, jnp.float32)
            srti[pl.ds(16 * t, 16)] = jnp.zeros((16,), jnp.int32)
            svm[t] = jnp.float32(NEG)
        smin[0] = jnp.float32(NEG)
        spos[0] = jnp.int32(0)

        def insert(xv, iv):
            def do():
                p = spos[0]
                t0 = p // 16
                s0 = t0 * 16
                lp = p - s0
                vec = topv[pl.ds(s0, 16)]
                nvec = jnp.where(iota == lp, xv, vec)
                topv[pl.ds(s0, 16)] = nvec
                ivec = topi[pl.ds(s0, 16)]
                topi[pl.ds(s0, 16)] = jnp.where(iota == lp, iv, ivec)
                svm[t0] = jnp.min(nvec)
                m = svm[0]
                pt = jnp.int32(0)
                for t in range(1, 7):
                    c = svm[t] < m
                    m = jnp.where(c, svm[t], m)
                    pt = jnp.where(c, jnp.int32(t), pt)
                lv = topv[pl.ds(pt * 16, 16)]
                lane = _ffs(lv == m, iota)
                smin[0] = m
                spos[0] = pt * 16 + lane

            pl.when(xv > smin[0])(do)

        def process_vec(x, ibase):
            maskb = x > smin[0]
            cnt = plsc.all_reduce_population_count(maskb)[0]

            def one(_, mk):
                lane = _ffs(mk > 0, iota)
                xv = jnp.max(
                    jnp.where(iota == lane, x, jnp.float32(NEG)))
                insert(xv, ibase + lane * C)
                return jnp.where(iota == lane, 0, mk)

            lax.fori_loop(0, cnt, one, maskb.astype(jnp.int32))

        def scan_plane(buf, c):
            def blk(g, _):
                q0 = g * 128
                vs = [buf[pl.ds(q0 + 16 * j, 16)] for j in range(8)]
                m1 = [jnp.maximum(vs[2 * j], vs[2 * j + 1]) for j in range(4)]
                m2 = [jnp.maximum(m1[0], m1[1]), jnp.maximum(m1[2], m1[3])]
                bm = jnp.maximum(m2[0], m2[1])
                cm = smin[0]

                def slow():
                    for p in range(4):
                        def pair(p=p):
                            for j in (2 * p, 2 * p + 1):
                                def hot(x=vs[j], j=j):
                                    process_vec(x, (q0 + 16 * j) * C + c)

                                pl.when(_any(vs[j] > smin[0]))(hot)

                        pl.when(_any(m1[p] > cm))(pair)

                pl.when(_any(bm > cm))(slow)
                return 0

            lax.fori_loop(0, NBLK, blk, 0)
            xt = jnp.where(iota < 8, jnp.float32(NEG),
                           buf[pl.ds(TAILQ, 16)])
            pl.when(_any(xt > smin[0]))(
                lambda: process_vec(xt, TAILQ * C + c))

        def start(c, buf, sem):
            pltpu.async_copy(logits_hbm.at[c, b, :], buf, sem)

        def wait(buf, sem):
            pltpu.make_async_copy(logits_hbm.at[0, b, :], buf, sem).wait()

        start(0, bufa, sema)

        def plane_body(c, _):
            def even():
                wait(bufa, sema)
                pl.when(c + 1 < C)(lambda: start(c + 1, bufb, semb))
                scan_plane(bufa, c)

            def odd():
                wait(bufb, semb)
                pl.when(c + 1 < C)(lambda: start(c + 1, bufa, sema))
                scan_plane(bufb, c)

            pl.when(c % 2 == 0)(even)
            pl.when(c % 2 == 1)(odd)
            return 0

        lax.fori_loop(0, C, plane_body, 0)

        topv[pl.ds(96, 16)] = jnp.where(iota < 4, topv[pl.ds(96, 16)],
                                        jnp.float32(NEG))

        def rank_body(r, _):
            tv = [topv[pl.ds(16 * t, 16)] for t in range(7)]
            mx = tv[0]
            for t in range(1, 7):
                mx = jnp.maximum(mx, tv[t])
            m = jnp.max(mx)
            ti = [topi[pl.ds(16 * t, 16)] for t in range(7)]
            cand = [jnp.where(tv[t] == m, ti[t], jnp.int32(BIGI))
                    for t in range(7)]
            cn = cand[0]
            for t in range(1, 7):
                cn = jnp.minimum(cn, cand[t])
            i = jnp.min(cn)
            for t in range(7):
                hit = (tv[t] == m) & (ti[t] == i)
                topv[pl.ds(16 * t, 16)] = jnp.where(hit, jnp.float32(NEG),
                                                    tv[t])
            s0 = (r // 16) * 16
            lp = r - s0
            sv = srtv[pl.ds(s0, 16)]
            srtv[pl.ds(s0, 16)] = jnp.where(iota == lp, m, sv)
            si = srti[pl.ds(s0, 16)]
            srti[pl.ds(s0, 16)] = jnp.where(iota == lp, i, si)
            return 0

        lax.fori_loop(0, TOPK, rank_body, 0)

        pltpu.sync_copy(scale_hbm.at[b], s16)
        for t in range(7):
            x = srtv[pl.ds(16 * t, 16)]
            ridx = srti[pl.ds(16 * t, 16)]
            rank = 16 * t + iota
            en = jnp.exp(jnp.where(x >= 0, -x, x))
            sig = jnp.where(x >= 0, 1.0 / (1.0 + en), en / (1.0 + en))
            keep = (sig > SCORE_THRESHOLD) & (rank < TOPK)
            q = ridx // C
            scv[pl.ds(16 * t, 16)] = jnp.where(keep, sig, jnp.float32(0.0))
            lbv[pl.ds(16 * t, 16)] = jnp.where(keep, ridx - q * C,
                                               jnp.int32(-1))
            qidx[pl.ds(16 * t, 16)] = q

        pltpu.sync_copy(boxes_hbm.at[b], boxtab)

        sv16 = s16[pl.ds(0, 16)]
        lm4 = iota % 4
        sgn = jnp.where(lm4 < 2, jnp.float32(-0.5), jnp.float32(0.5))

        def box_body(g, _):
            slot = g * 4 + iota // 4
            qg = plsc.load_gather(qidx, [slot])
            acol = iota % 2
            a = plsc.load_gather(boxtab, [acol, qg])
            bb = plsc.load_gather(boxtab, [acol + 2, qg])
            xyxy = (a + sgn * bb) * sv16
            xyv[pl.ds(g * 16, 16)] = jnp.where(lm4 < 2, xyxy, bb * sv16)
            return 0

        lax.fori_loop(0, 28, box_body, 0)

        pltpu.sync_copy(scv, out_s.at[b])
        pltpu.sync_copy(lbv, out_l.at[b])
        pltpu.sync_copy(xyv, out_x.at[b])

    return body(logits_flat, boxes_flat, scale16)


def kernel(pred_logits, pred_boxes, target_sizes, image_ids):
    logits_t = jnp.transpose(pred_logits, (2, 0, 1))
    boxes_t = jnp.transpose(pred_boxes, (0, 2, 1))
    ts = target_sizes.astype(jnp.float32)
    scale16 = jnp.tile(jnp.stack([ts[:, 1], ts[:, 0]], axis=-1), (1, 8))
    out_s, out_l, out_x = _sc_call(logits_t, boxes_t, scale16)
    scores = out_s[:, :TOPK]
    labels = out_l[:, :TOPK]
    xywh = out_x.reshape(B, KPAD, 4)[:, :TOPK, :]
    det_image_ids = jnp.broadcast_to(image_ids[:, None], (B, TOPK))
    return scores, labels, xywh, det_image_ids
